# Initial kernel scaffold; baseline (speedup 1.0000x reference)
import jax
import jax.numpy as jnp
from jax.experimental import pallas as pl


def kernel(x, edge_index, edge_type, comp1, basis1, root1, bias1, comp2, basis2, root2, bias2, Wc, bc):
    raise NotImplementedError("write your pallas kernel here")


# speedup vs baseline: 30.0560x; 30.0560x over previous
import jax
import jax.numpy as jnp
from jax import lax
from jax.experimental import pallas as pl
from jax.experimental.pallas import tpu as pltpu
from jax.experimental.pallas import tpu_sc as plsc

_NUM_CORES = 2
_NUM_SUBCORES = 16
_NW = _NUM_CORES * _NUM_SUBCORES
_L = 16
_CH = 80
_ZROWS = 8


def _pad_rows(n):
    rpt = -(-n // _NUM_SUBCORES)
    rpt = -(-rpt // 8) * 8
    return rpt, rpt * _NUM_SUBCORES

_HP = lax.Precision.HIGHEST


def _mesh():
    return plsc.VectorSubcoreMesh(
        core_axis_name="c", subcore_axis_name="s",
        num_cores=_NUM_CORES, num_subcores=_NUM_SUBCORES)


def _assemble_body(c1_ref, b1_ref, c2_ref, b2_ref, w1_ref, w2_ref):
    w1_ref[...] = jnp.dot(c1_ref[...], b1_ref[...],
                          preferred_element_type=jnp.float32, precision=_HP)
    w2_ref[...] = jnp.dot(c2_ref[...], b2_ref[...],
                          preferred_element_type=jnp.float32, precision=_HP)


def _assemble(c1, b1f, c2, b2f):
    r, _ = c1.shape
    return pl.pallas_call(
        _assemble_body,
        out_shape=[jax.ShapeDtypeStruct((r, b1f.shape[1]), jnp.float32),
                   jax.ShapeDtypeStruct((r, b2f.shape[1]), jnp.float32)],
    )(c1, b1f, c2, b2f)


def _inv_body(c_ref, o_ref):
    o_ref[...] = 1.0 / jnp.maximum(c_ref[0] + c_ref[1], 1.0)


def _inv(cnt3):
    _, m, k = cnt3.shape
    return pl.pallas_call(
        _inv_body,
        out_shape=jax.ShapeDtypeStruct((m, k), jnp.float32),
    )(cnt3)


def _layer_in_body(x_ref, wcat_ref, root_ref, bias_ref, y_ref, p_ref):
    xv = x_ref[...]
    y_ref[...] = jnp.dot(xv, wcat_ref[...],
                         preferred_element_type=jnp.float32, precision=_HP)
    p_ref[...] = jnp.dot(xv, root_ref[...],
                         preferred_element_type=jnp.float32,
                         precision=_HP) + bias_ref[...]


def _layer_in(x, wcat, root, bias):
    n, in_c = x.shape
    rd = wcat.shape[1]
    hid = root.shape[1]
    bn = 512
    return pl.pallas_call(
        _layer_in_body,
        grid=(pl.cdiv(n, bn),),
        in_specs=[pl.BlockSpec((bn, in_c), lambda i: (i, 0)),
                  pl.BlockSpec((in_c, rd), lambda i: (0, 0)),
                  pl.BlockSpec((in_c, hid), lambda i: (0, 0)),
                  pl.BlockSpec((1, hid), lambda i: (0, 0))],
        out_specs=[pl.BlockSpec((bn, rd), lambda i: (i, 0)),
                   pl.BlockSpec((bn, hid), lambda i: (i, 0))],
        out_shape=[jax.ShapeDtypeStruct((n, rd), jnp.float32),
                   jax.ShapeDtypeStruct((n, hid), jnp.float32)],
    )(x, wcat, root, bias)


def _layer_mid_body(p_ref, m0_ref, m1_ref, wcat_ref, root_ref, bias_ref,
                    y_ref, p2_ref):
    h = jnp.maximum(p_ref[...] + m0_ref[...] + m1_ref[...], 0.0)
    y_ref[...] = jnp.dot(h, wcat_ref[...],
                         preferred_element_type=jnp.float32, precision=_HP)
    p2_ref[...] = jnp.dot(h, root_ref[...],
                          preferred_element_type=jnp.float32,
                          precision=_HP) + bias_ref[...]


def _layer_mid(p, m0, m1, wcat, root, bias):
    n, hid = p.shape
    rd = wcat.shape[1]
    out_c = root.shape[1]
    bn = 512
    return pl.pallas_call(
        _layer_mid_body,
        grid=(pl.cdiv(n, bn),),
        in_specs=[pl.BlockSpec((bn, hid), lambda i: (i, 0)),
                  pl.BlockSpec((bn, hid), lambda i: (i, 0)),
                  pl.BlockSpec((bn, hid), lambda i: (i, 0)),
                  pl.BlockSpec((hid, rd), lambda i: (0, 0)),
                  pl.BlockSpec((hid, out_c), lambda i: (0, 0)),
                  pl.BlockSpec((1, out_c), lambda i: (0, 0))],
        out_specs=[pl.BlockSpec((bn, rd), lambda i: (i, 0)),
                   pl.BlockSpec((bn, out_c), lambda i: (i, 0))],
        out_shape=[jax.ShapeDtypeStruct((n, rd), jnp.float32),
                   jax.ShapeDtypeStruct((n, out_c), jnp.float32)],
    )(p, m0, m1, wcat, root, bias)


def _final_body(p_ref, m0_ref, m1_ref, wc_ref, bc_ref, o_ref):
    h = p_ref[...] + m0_ref[...] + m1_ref[...]
    o_ref[...] = jnp.dot(h, wc_ref[...],
                         preferred_element_type=jnp.float32,
                         precision=_HP) + bc_ref[...]


def _final(p, m0, m1, wc, bc):
    n, out_c = p.shape
    k = wc.shape[1]
    bn = 512
    return pl.pallas_call(
        _final_body,
        grid=(pl.cdiv(n, bn),),
        in_specs=[pl.BlockSpec((bn, out_c), lambda i: (i, 0)),
                  pl.BlockSpec((bn, out_c), lambda i: (i, 0)),
                  pl.BlockSpec((bn, out_c), lambda i: (i, 0)),
                  pl.BlockSpec((out_c, k), lambda i: (0, 0)),
                  pl.BlockSpec((1, k), lambda i: (0, 0))],
        out_specs=pl.BlockSpec((bn, k), lambda i: (i, 0)),
        out_shape=jax.ShapeDtypeStruct((n, k), jnp.float32),
    )(p, m0, m1, wc, bc)


def _count(dst, typ, n_nodes, n_rel):
    e = dst.shape[0]
    rows_per_tile, n_pad = _pad_rows(n_nodes)
    edges_per_tile = e // _NW
    n_chunks = edges_per_tile // _CH
    ngrp = _CH // _L
    ccol = n_rel // _L
    w = 128

    def body(dst_hbm, typ_hbm, out_hbm, dstb, typb, oh, zb, acc):
        core = lax.axis_index("c")
        sub = lax.axis_index("s")
        wid = core * _NUM_SUBCORES + sub
        zeros16 = jnp.zeros((_L,), jnp.float32)
        ones16 = jnp.ones((_L,), jnp.float32)
        iota16 = lax.iota(jnp.int32, _L)
        for i in range(_ZROWS):
            for cc in range(w // _L):
                zb[i, pl.ds(cc * _L, _L)] = zeros16
        for i in range(_CH):
            for cc in range(w // _L):
                oh[i, pl.ds(cc * _L, _L)] = zeros16

        def zloop(i, carry):
            pltpu.sync_copy(
                zb, acc.at[pl.ds(sub * rows_per_tile + i * _ZROWS, _ZROWS)])
            return carry
        lax.fori_loop(0, rows_per_tile // _ZROWS, zloop, 0)
        plsc.subcore_barrier()

        ebase = wid * edges_per_tile

        def chunk(i, carry):
            base = ebase + i * _CH
            pltpu.sync_copy(dst_hbm.at[pl.ds(base, _CH)], dstb)
            pltpu.sync_copy(typ_hbm.at[pl.ds(base, _CH)], typb)

            def mark(g, c2):
                t16 = typb[pl.ds(g * _L, _L)]
                for j in range(_L):
                    e = g * _L + j
                    ts = jnp.full((_L,), t16[j], jnp.int32)
                    for cc in range(ccol):
                        oh[e, pl.ds(cc * _L, _L)] = jnp.where(
                            iota16 + cc * _L == ts, ones16, zeros16)
                return c2
            lax.fori_loop(0, ngrp, mark, 0)
            pltpu.sync_copy(oh, acc.at[dstb], add=True)
            return carry
        lax.fori_loop(0, n_chunks, chunk, 0)
        plsc.subcore_barrier()

        row0 = sub * rows_per_tile
        pltpu.sync_copy(
            acc.at[pl.ds(row0, rows_per_tile)],
            out_hbm.at[pl.ds(core * n_pad + row0, rows_per_tile)])

    return pl.kernel(
        body,
        out_type=jax.ShapeDtypeStruct((2 * n_pad, w), jnp.float32),
        mesh=_mesh(),
        scratch_types=[
            pltpu.VMEM((_CH,), jnp.int32),
            pltpu.VMEM((_CH,), jnp.int32),
            pltpu.VMEM((_CH, w), jnp.float32),
            pltpu.VMEM((_ZROWS, w), jnp.float32),
            pltpu.VMEM_SHARED((n_pad, w), jnp.float32),
        ],
    )(dst, typ)


def _edge_pass(ytab, src, dst, typ, inv, n_nodes, n_rel, d):
    e = src.shape[0]
    rows_per_tile, n_pad = _pad_rows(n_nodes)
    edges_per_tile = e // _NW
    n_chunks = edges_per_tile // _CH
    ngrp = _CH // _L
    dcol = d // _L

    def body(ytab_hbm, src_hbm, dst_hbm, typ_hbm, inv_hbm, out_hbm,
             srcb, dstb, typb, keyb, rows, invr, zb, acc):
        core = lax.axis_index("c")
        sub = lax.axis_index("s")
        wid = core * _NUM_SUBCORES + sub
        zeros16 = jnp.zeros((_L,), jnp.float32)
        iota16 = lax.iota(jnp.int32, _L)
        for i in range(_ZROWS):
            for cc in range(dcol):
                zb[i, pl.ds(cc * _L, _L)] = zeros16

        def zloop(i, carry):
            pltpu.sync_copy(
                zb, acc.at[pl.ds(sub * rows_per_tile + i * _ZROWS, _ZROWS)])
            return carry
        lax.fori_loop(0, rows_per_tile // _ZROWS, zloop, 0)
        plsc.subcore_barrier()

        ebase = wid * edges_per_tile

        def chunk(i, carry):
            base = ebase + i * _CH
            pltpu.sync_copy(src_hbm.at[pl.ds(base, _CH)], srcb)
            pltpu.sync_copy(dst_hbm.at[pl.ds(base, _CH)], dstb)
            pltpu.sync_copy(typ_hbm.at[pl.ds(base, _CH)], typb)

            def kgrp(g, c2):
                s16 = srcb[pl.ds(g * _L, _L)]
                t16 = typb[pl.ds(g * _L, _L)]
                keyb[pl.ds(g * _L, _L)] = s16 * n_rel + t16
                return c2
            lax.fori_loop(0, ngrp, kgrp, 0)

            pltpu.sync_copy(ytab_hbm.at[keyb], rows)
            pltpu.sync_copy(inv_hbm.at[dstb], invr)

            def sgrp(g, c2):
                t16 = typb[pl.ds(g * _L, _L)]
                for j in range(_L):
                    e = g * _L + j
                    w = invr[e, pl.ds(t16[j], _L)][0]
                    ws = jnp.full((_L,), w, jnp.float32)
                    for cc in range(dcol):
                        rows[e, pl.ds(cc * _L, _L)] = (
                            rows[e, pl.ds(cc * _L, _L)] * ws)
                return c2
            lax.fori_loop(0, ngrp, sgrp, 0)

            pltpu.sync_copy(rows, acc.at[dstb], add=True)
            return carry
        lax.fori_loop(0, n_chunks, chunk, 0)
        plsc.subcore_barrier()

        row0 = sub * rows_per_tile
        pltpu.sync_copy(
            acc.at[pl.ds(row0, rows_per_tile)],
            out_hbm.at[pl.ds(core * n_pad + row0, rows_per_tile)])

    return pl.kernel(
        body,
        out_type=jax.ShapeDtypeStruct((2 * n_pad, d), jnp.float32),
        mesh=_mesh(),
        scratch_types=[
            pltpu.VMEM((_CH,), jnp.int32),
            pltpu.VMEM((_CH,), jnp.int32),
            pltpu.VMEM((_CH,), jnp.int32),
            pltpu.VMEM((_CH,), jnp.int32),
            pltpu.VMEM((_CH, d), jnp.float32),
            pltpu.VMEM((_CH, 128), jnp.float32),
            pltpu.VMEM((_ZROWS, d), jnp.float32),
            pltpu.VMEM_SHARED((n_pad, d), jnp.float32),
        ],
    )(ytab, src, dst, typ, inv)


def kernel(x, edge_index, edge_type, comp1, basis1, root1, bias1,
           comp2, basis2, root2, bias2, Wc, bc):
    n, in_c = x.shape
    hid = root1.shape[1]
    out_c = root2.shape[1]
    r, nb = comp1.shape
    ncls = Wc.shape[1]

    src = edge_index[0]
    dst = edge_index[1]

    w1f, w2f = _assemble(comp1, basis1.reshape(nb, in_c * hid),
                         comp2, basis2.reshape(nb, hid * out_c))
    wcat1 = w1f.reshape(r, in_c, hid).transpose(1, 0, 2).reshape(in_c, r * hid)
    wcat2 = w2f.reshape(r, hid, out_c).transpose(1, 0, 2).reshape(hid, r * out_c)

    _, n_pad = _pad_rows(n)
    cnt = _count(dst, edge_type, n, r)
    inv = _inv(cnt.reshape(2, n_pad, 128))

    y1, p1 = _layer_in(x, wcat1, root1, bias1.reshape(1, hid))
    m1 = _edge_pass(y1.reshape(n * r, hid), src, dst, edge_type, inv,
                    n, r, hid)

    y2, p2 = _layer_mid(p1, m1[:n], m1[n_pad:n_pad + n], wcat2, root2,
                        bias2.reshape(1, out_c))
    m2 = _edge_pass(y2.reshape(n * r, out_c), src, dst, edge_type, inv,
                    n, r, out_c)

    wc_pad = jnp.zeros((out_c, 128), jnp.float32).at[:, :ncls].set(Wc)
    bc_pad = jnp.zeros((1, 128), jnp.float32).at[0, :ncls].set(bc)
    of = _final(p2, m2[:n], m2[n_pad:n_pad + n], wc_pad, bc_pad)
    return of[:, :ncls]


# speedup vs baseline: 42.5650x; 1.4162x over previous
import jax
import jax.numpy as jnp
from jax import lax
from jax.experimental import pallas as pl
from jax.experimental.pallas import tpu as pltpu
from jax.experimental.pallas import tpu_sc as plsc

_NUM_CORES = 2
_NUM_SUBCORES = 16
_NW = _NUM_CORES * _NUM_SUBCORES
_L = 16
_CH = 80
_ZROWS = 8


def _pad_rows(n):
    rpt = -(-n // _NUM_SUBCORES)
    rpt = -(-rpt // 8) * 8
    return rpt, rpt * _NUM_SUBCORES

_HP = lax.Precision.HIGHEST


def _mesh():
    return plsc.VectorSubcoreMesh(
        core_axis_name="c", subcore_axis_name="s",
        num_cores=_NUM_CORES, num_subcores=_NUM_SUBCORES)


def _assemble_body(c1_ref, b1_ref, c2_ref, b2_ref, w1_ref, w2_ref):
    w1_ref[...] = jnp.dot(c1_ref[...], b1_ref[...],
                          preferred_element_type=jnp.float32, precision=_HP)
    w2_ref[...] = jnp.dot(c2_ref[...], b2_ref[...],
                          preferred_element_type=jnp.float32, precision=_HP)


def _assemble(c1, b1f, c2, b2f):
    r, _ = c1.shape
    return pl.pallas_call(
        _assemble_body,
        out_shape=[jax.ShapeDtypeStruct((r, b1f.shape[1]), jnp.float32),
                   jax.ShapeDtypeStruct((r, b2f.shape[1]), jnp.float32)],
    )(c1, b1f, c2, b2f)


def _inv_body(c_ref, o_ref):
    o_ref[...] = 1.0 / jnp.maximum(c_ref[0] + c_ref[1], 1.0)


def _inv(cnt3):
    _, m, k = cnt3.shape
    return pl.pallas_call(
        _inv_body,
        out_shape=jax.ShapeDtypeStruct((m, k), jnp.float32),
    )(cnt3)


def _layer_in_body(x_ref, wcat_ref, root_ref, bias_ref, y_ref, p_ref):
    xv = x_ref[...]
    y_ref[...] = jnp.dot(xv, wcat_ref[...],
                         preferred_element_type=jnp.float32, precision=_HP)
    p_ref[...] = jnp.dot(xv, root_ref[...],
                         preferred_element_type=jnp.float32,
                         precision=_HP) + bias_ref[...]


def _layer_in(x, wcat, root, bias):
    n, in_c = x.shape
    rd = wcat.shape[1]
    hid = root.shape[1]
    bn = 512
    return pl.pallas_call(
        _layer_in_body,
        grid=(pl.cdiv(n, bn),),
        in_specs=[pl.BlockSpec((bn, in_c), lambda i: (i, 0)),
                  pl.BlockSpec((in_c, rd), lambda i: (0, 0)),
                  pl.BlockSpec((in_c, hid), lambda i: (0, 0)),
                  pl.BlockSpec((1, hid), lambda i: (0, 0))],
        out_specs=[pl.BlockSpec((bn, rd), lambda i: (i, 0)),
                   pl.BlockSpec((bn, hid), lambda i: (i, 0))],
        out_shape=[jax.ShapeDtypeStruct((n, rd), jnp.float32),
                   jax.ShapeDtypeStruct((n, hid), jnp.float32)],
    )(x, wcat, root, bias)


def _layer_mid_body(p_ref, m0_ref, m1_ref, wcat_ref, root_ref, bias_ref,
                    y_ref, p2_ref):
    h = jnp.maximum(p_ref[...] + m0_ref[...] + m1_ref[...], 0.0)
    y_ref[...] = jnp.dot(h, wcat_ref[...],
                         preferred_element_type=jnp.float32, precision=_HP)
    p2_ref[...] = jnp.dot(h, root_ref[...],
                          preferred_element_type=jnp.float32,
                          precision=_HP) + bias_ref[...]


def _layer_mid(p, m0, m1, wcat, root, bias):
    n, hid = p.shape
    rd = wcat.shape[1]
    out_c = root.shape[1]
    bn = 512
    return pl.pallas_call(
        _layer_mid_body,
        grid=(pl.cdiv(n, bn),),
        in_specs=[pl.BlockSpec((bn, hid), lambda i: (i, 0)),
                  pl.BlockSpec((bn, hid), lambda i: (i, 0)),
                  pl.BlockSpec((bn, hid), lambda i: (i, 0)),
                  pl.BlockSpec((hid, rd), lambda i: (0, 0)),
                  pl.BlockSpec((hid, out_c), lambda i: (0, 0)),
                  pl.BlockSpec((1, out_c), lambda i: (0, 0))],
        out_specs=[pl.BlockSpec((bn, rd), lambda i: (i, 0)),
                   pl.BlockSpec((bn, out_c), lambda i: (i, 0))],
        out_shape=[jax.ShapeDtypeStruct((n, rd), jnp.float32),
                   jax.ShapeDtypeStruct((n, out_c), jnp.float32)],
    )(p, m0, m1, wcat, root, bias)


def _final_body(p_ref, m0_ref, m1_ref, wc_ref, bc_ref, o_ref):
    h = p_ref[...] + m0_ref[...] + m1_ref[...]
    o_ref[...] = jnp.dot(h, wc_ref[...],
                         preferred_element_type=jnp.float32,
                         precision=_HP) + bc_ref[...]


def _final(p, m0, m1, wc, bc):
    n, out_c = p.shape
    k = wc.shape[1]
    bn = 512
    return pl.pallas_call(
        _final_body,
        grid=(pl.cdiv(n, bn),),
        in_specs=[pl.BlockSpec((bn, out_c), lambda i: (i, 0)),
                  pl.BlockSpec((bn, out_c), lambda i: (i, 0)),
                  pl.BlockSpec((bn, out_c), lambda i: (i, 0)),
                  pl.BlockSpec((out_c, k), lambda i: (0, 0)),
                  pl.BlockSpec((1, k), lambda i: (0, 0))],
        out_specs=pl.BlockSpec((bn, k), lambda i: (i, 0)),
        out_shape=jax.ShapeDtypeStruct((n, k), jnp.float32),
    )(p, m0, m1, wc, bc)


def _count(dst, typ, n_nodes, n_rel):
    e = dst.shape[0]
    rows_per_tile, n_pad = _pad_rows(n_nodes)
    edges_per_tile = e // _NW
    n_chunks = edges_per_tile // _CH
    ngrp = _CH // _L
    ccol = n_rel // _L
    w = 128

    def body(dst_hbm, typ_hbm, out_hbm, dstb, typb, oh, zb, acc):
        core = lax.axis_index("c")
        sub = lax.axis_index("s")
        wid = core * _NUM_SUBCORES + sub
        zeros16 = jnp.zeros((_L,), jnp.float32)
        ones16 = jnp.ones((_L,), jnp.float32)
        iota16 = lax.iota(jnp.int32, _L)
        for i in range(_ZROWS):
            for cc in range(w // _L):
                zb[i, pl.ds(cc * _L, _L)] = zeros16
        for i in range(_CH):
            for cc in range(w // _L):
                oh[i, pl.ds(cc * _L, _L)] = zeros16

        def zloop(i, carry):
            pltpu.sync_copy(
                zb, acc.at[pl.ds(sub * rows_per_tile + i * _ZROWS, _ZROWS)])
            return carry
        lax.fori_loop(0, rows_per_tile // _ZROWS, zloop, 0)
        plsc.subcore_barrier()

        ebase = wid * edges_per_tile

        def chunk(i, carry):
            base = ebase + i * _CH
            pltpu.sync_copy(dst_hbm.at[pl.ds(base, _CH)], dstb)
            pltpu.sync_copy(typ_hbm.at[pl.ds(base, _CH)], typb)

            def mark(g, c2):
                t16 = typb[pl.ds(g * _L, _L)]
                for j in range(_L):
                    e = g * _L + j
                    ts = jnp.full((_L,), t16[j], jnp.int32)
                    for cc in range(ccol):
                        oh[e, pl.ds(cc * _L, _L)] = jnp.where(
                            iota16 + cc * _L == ts, ones16, zeros16)
                return c2
            lax.fori_loop(0, ngrp, mark, 0)
            pltpu.sync_copy(oh, acc.at[dstb], add=True)
            return carry
        lax.fori_loop(0, n_chunks, chunk, 0)
        plsc.subcore_barrier()

        row0 = sub * rows_per_tile
        pltpu.sync_copy(
            acc.at[pl.ds(row0, rows_per_tile)],
            out_hbm.at[pl.ds(core * n_pad + row0, rows_per_tile)])

    return pl.kernel(
        body,
        out_type=jax.ShapeDtypeStruct((2 * n_pad, w), jnp.float32),
        mesh=_mesh(),
        scratch_types=[
            pltpu.VMEM((_CH,), jnp.int32),
            pltpu.VMEM((_CH,), jnp.int32),
            pltpu.VMEM((_CH, w), jnp.float32),
            pltpu.VMEM((_ZROWS, w), jnp.float32),
            pltpu.VMEM_SHARED((n_pad, w), jnp.float32),
        ],
    )(dst, typ)


def _edge_pass(ytab, src, dst, typ, inv, n_nodes, n_rel, d):
    e = src.shape[0]
    rows_per_tile, n_pad = _pad_rows(n_nodes)
    edges_per_tile = e // _NW
    n_chunks = edges_per_tile // _CH
    n_pairs = (n_chunks - 1) // 2
    assert n_chunks == 2 * n_pairs + 1 and n_chunks >= 3
    ngrp = _CH // _L
    dcol = d // _L

    def body(ytab_hbm, src_hbm, dst_hbm, typ_hbm, inv_hbm, out_hbm,
             srcb0, srcb1, dstb0, dstb1, typb0, typb1, keyb0, keyb1,
             adst0, adst1, rows0, rows1, invr0, invr1, zb, acc,
             sl0, sl1, sg, sa0, sa1):
        core = lax.axis_index("c")
        sub = lax.axis_index("s")
        wid = core * _NUM_SUBCORES + sub
        srcbs = (srcb0, srcb1)
        dstbs = (dstb0, dstb1)
        typbs = (typb0, typb1)
        keybs = (keyb0, keyb1)
        adsts = (adst0, adst1)
        rowss = (rows0, rows1)
        invrs = (invr0, invr1)
        sls = (sl0, sl1)
        sas = (sa0, sa1)
        zeros16 = jnp.zeros((_L,), jnp.float32)
        for i in range(_ZROWS):
            for cc in range(dcol):
                zb[i, pl.ds(cc * _L, _L)] = zeros16

        def zloop(i, carry):
            pltpu.sync_copy(
                zb, acc.at[pl.ds(sub * rows_per_tile + i * _ZROWS, _ZROWS)])
            return carry
        lax.fori_loop(0, rows_per_tile // _ZROWS, zloop, 0)
        plsc.subcore_barrier()

        ebase = wid * edges_per_tile

        def issue_load(i, b):
            base = ebase + i * _CH
            pltpu.async_copy(src_hbm.at[pl.ds(base, _CH)], srcbs[b], sls[b])
            pltpu.async_copy(dst_hbm.at[pl.ds(base, _CH)], dstbs[b], sls[b])
            pltpu.async_copy(typ_hbm.at[pl.ds(base, _CH)], typbs[b], sls[b])

        def wait_load(b):
            pltpu.make_async_copy(
                src_hbm.at[pl.ds(ebase, _CH)], srcbs[b], sls[b]).wait()
            pltpu.make_async_copy(
                dst_hbm.at[pl.ds(ebase, _CH)], dstbs[b], sls[b]).wait()
            pltpu.make_async_copy(
                typ_hbm.at[pl.ds(ebase, _CH)], typbs[b], sls[b]).wait()

        def wait_acc(b):
            pltpu.make_async_copy(rowss[b], acc.at[adsts[b]], sas[b]).wait()

        def make_kgrp(b):
            def kgrp(g, c2):
                s16 = srcbs[b][pl.ds(g * _L, _L)]
                t16 = typbs[b][pl.ds(g * _L, _L)]
                keybs[b][pl.ds(g * _L, _L)] = s16 * n_rel + t16
                adsts[b][pl.ds(g * _L, _L)] = dstbs[b][pl.ds(g * _L, _L)]
                return c2
            return kgrp

        def make_sgrp(b):
            rows, invr, typb = rowss[b], invrs[b], typbs[b]

            def sgrp(g, c2):
                t16 = typb[pl.ds(g * _L, _L)]
                for j in range(_L):
                    ee = g * _L + j
                    w = invr[ee, pl.ds(t16[j], _L)][0]
                    ws = jnp.full((_L,), w, jnp.float32)
                    for cc in range(dcol):
                        rows[ee, pl.ds(cc * _L, _L)] = (
                            rows[ee, pl.ds(cc * _L, _L)] * ws)
                return c2
            return sgrp

        def issue_gather(b):
            pltpu.async_copy(ytab_hbm.at[keybs[b]], rowss[b], sg)
            pltpu.async_copy(inv_hbm.at[adsts[b]], invrs[b], sg)

        def wait_gather(b):
            pltpu.make_async_copy(ytab_hbm.at[keybs[b]], rowss[b], sg).wait()
            pltpu.make_async_copy(inv_hbm.at[adsts[b]], invrs[b], sg).wait()

        def process(i, b, steady):
            wait_load(b)
            if steady:
                pl.when(i >= 2)(lambda: wait_acc(b))
            lax.fori_loop(0, ngrp, make_kgrp(b), 0)
            if steady:
                pl.when(i + 1 < n_chunks)(lambda: issue_load(i + 1, 1 - b))
            else:
                issue_load(i + 1, 1 - b)
            issue_gather(b)
            wait_gather(b)
            lax.fori_loop(0, ngrp, make_sgrp(b), 0)
            pltpu.async_copy(rowss[b], acc.at[adsts[b]], sas[b], add=True)

        issue_load(0, 0)
        process(0, 0, False)

        def pair(ii, carry):
            process(2 * ii + 1, 1, True)
            process(2 * ii + 2, 0, True)
            return carry
        lax.fori_loop(0, n_pairs, pair, 0)

        wait_acc(0)
        wait_acc(1)
        plsc.subcore_barrier()

        row0 = sub * rows_per_tile
        pltpu.sync_copy(
            acc.at[pl.ds(row0, rows_per_tile)],
            out_hbm.at[pl.ds(core * n_pad + row0, rows_per_tile)])

    return pl.kernel(
        body,
        out_type=jax.ShapeDtypeStruct((2 * n_pad, d), jnp.float32),
        mesh=_mesh(),
        scratch_types=[
            pltpu.VMEM((_CH,), jnp.int32),
            pltpu.VMEM((_CH,), jnp.int32),
            pltpu.VMEM((_CH,), jnp.int32),
            pltpu.VMEM((_CH,), jnp.int32),
            pltpu.VMEM((_CH,), jnp.int32),
            pltpu.VMEM((_CH,), jnp.int32),
            pltpu.VMEM((_CH,), jnp.int32),
            pltpu.VMEM((_CH,), jnp.int32),
            pltpu.VMEM((_CH,), jnp.int32),
            pltpu.VMEM((_CH,), jnp.int32),
            pltpu.VMEM((_CH, d), jnp.float32),
            pltpu.VMEM((_CH, d), jnp.float32),
            pltpu.VMEM((_CH, 128), jnp.float32),
            pltpu.VMEM((_CH, 128), jnp.float32),
            pltpu.VMEM((_ZROWS, d), jnp.float32),
            pltpu.VMEM_SHARED((n_pad, d), jnp.float32),
            pltpu.SemaphoreType.DMA,
            pltpu.SemaphoreType.DMA,
            pltpu.SemaphoreType.DMA,
            pltpu.SemaphoreType.DMA,
            pltpu.SemaphoreType.DMA,
        ],
    )(ytab, src, dst, typ, inv)


def kernel(x, edge_index, edge_type, comp1, basis1, root1, bias1,
           comp2, basis2, root2, bias2, Wc, bc):
    n, in_c = x.shape
    hid = root1.shape[1]
    out_c = root2.shape[1]
    r, nb = comp1.shape
    ncls = Wc.shape[1]

    src = edge_index[0]
    dst = edge_index[1]

    w1f, w2f = _assemble(comp1, basis1.reshape(nb, in_c * hid),
                         comp2, basis2.reshape(nb, hid * out_c))
    wcat1 = w1f.reshape(r, in_c, hid).transpose(1, 0, 2).reshape(in_c, r * hid)
    wcat2 = w2f.reshape(r, hid, out_c).transpose(1, 0, 2).reshape(hid, r * out_c)

    _, n_pad = _pad_rows(n)
    cnt = _count(dst, edge_type, n, r)
    inv = _inv(cnt.reshape(2, n_pad, 128))

    y1, p1 = _layer_in(x, wcat1, root1, bias1.reshape(1, hid))
    m1 = _edge_pass(y1.reshape(n * r, hid), src, dst, edge_type, inv,
                    n, r, hid)

    y2, p2 = _layer_mid(p1, m1[:n], m1[n_pad:n_pad + n], wcat2, root2,
                        bias2.reshape(1, out_c))
    m2 = _edge_pass(y2.reshape(n * r, out_c), src, dst, edge_type, inv,
                    n, r, out_c)

    wc_pad = jnp.zeros((out_c, 128), jnp.float32).at[:, :ncls].set(Wc)
    bc_pad = jnp.zeros((1, 128), jnp.float32).at[0, :ncls].set(bc)
    of = _final(p2, m2[:n], m2[n_pad:n_pad + n], wc_pad, bc_pad)
    return of[:, :ncls]


# speedup vs baseline: 48.9636x; 1.1503x over previous
import jax
import jax.numpy as jnp
from jax import lax
from jax.experimental import pallas as pl
from jax.experimental.pallas import tpu as pltpu
from jax.experimental.pallas import tpu_sc as plsc

_NUM_CORES = 2
_NUM_SUBCORES = 16
_NW = _NUM_CORES * _NUM_SUBCORES
_L = 16
_CH = 80
_ZROWS = 8


def _pad_rows(n):
    rpt = -(-n // _NUM_SUBCORES)
    rpt = -(-rpt // 8) * 8
    return rpt, rpt * _NUM_SUBCORES

_HP = lax.Precision.DEFAULT


def _mesh():
    return plsc.VectorSubcoreMesh(
        core_axis_name="c", subcore_axis_name="s",
        num_cores=_NUM_CORES, num_subcores=_NUM_SUBCORES)


def _assemble_body(c1_ref, b1_ref, c2_ref, b2_ref, w1_ref, w2_ref):
    w1_ref[...] = jnp.dot(c1_ref[...], b1_ref[...],
                          preferred_element_type=jnp.float32, precision=_HP)
    w2_ref[...] = jnp.dot(c2_ref[...], b2_ref[...],
                          preferred_element_type=jnp.float32, precision=_HP)


def _assemble(c1, b1f, c2, b2f):
    r, _ = c1.shape
    return pl.pallas_call(
        _assemble_body,
        out_shape=[jax.ShapeDtypeStruct((r, b1f.shape[1]), jnp.float32),
                   jax.ShapeDtypeStruct((r, b2f.shape[1]), jnp.float32)],
    )(c1, b1f, c2, b2f)


def _inv_body(c_ref, o_ref):
    o_ref[...] = 1.0 / jnp.maximum(c_ref[0] + c_ref[1], 1.0)


def _inv(cnt3):
    _, m, k = cnt3.shape
    return pl.pallas_call(
        _inv_body,
        out_shape=jax.ShapeDtypeStruct((m, k), jnp.float32),
    )(cnt3)


def _layer_in_body(x_ref, wcat_ref, root_ref, bias_ref, y_ref, p_ref):
    xv = x_ref[...]
    y_ref[...] = jnp.dot(xv, wcat_ref[...],
                         preferred_element_type=jnp.float32, precision=_HP)
    p_ref[...] = jnp.dot(xv, root_ref[...],
                         preferred_element_type=jnp.float32,
                         precision=_HP) + bias_ref[...]


def _layer_in(x, wcat, root, bias):
    n, in_c = x.shape
    rd = wcat.shape[1]
    hid = root.shape[1]
    bn = 512
    return pl.pallas_call(
        _layer_in_body,
        grid=(pl.cdiv(n, bn),),
        in_specs=[pl.BlockSpec((bn, in_c), lambda i: (i, 0)),
                  pl.BlockSpec((in_c, rd), lambda i: (0, 0)),
                  pl.BlockSpec((in_c, hid), lambda i: (0, 0)),
                  pl.BlockSpec((1, hid), lambda i: (0, 0))],
        out_specs=[pl.BlockSpec((bn, rd), lambda i: (i, 0)),
                   pl.BlockSpec((bn, hid), lambda i: (i, 0))],
        out_shape=[jax.ShapeDtypeStruct((n, rd), jnp.float32),
                   jax.ShapeDtypeStruct((n, hid), jnp.float32)],
    )(x, wcat, root, bias)


def _layer_mid_body(p_ref, m0_ref, m1_ref, wcat_ref, root_ref, bias_ref,
                    y_ref, p2_ref):
    h = jnp.maximum(p_ref[...] + m0_ref[...] + m1_ref[...], 0.0)
    y_ref[...] = jnp.dot(h, wcat_ref[...],
                         preferred_element_type=jnp.float32, precision=_HP)
    p2_ref[...] = jnp.dot(h, root_ref[...],
                          preferred_element_type=jnp.float32,
                          precision=_HP) + bias_ref[...]


def _layer_mid(p, m0, m1, wcat, root, bias):
    n, hid = p.shape
    rd = wcat.shape[1]
    out_c = root.shape[1]
    bn = 512
    return pl.pallas_call(
        _layer_mid_body,
        grid=(pl.cdiv(n, bn),),
        in_specs=[pl.BlockSpec((bn, hid), lambda i: (i, 0)),
                  pl.BlockSpec((bn, hid), lambda i: (i, 0)),
                  pl.BlockSpec((bn, hid), lambda i: (i, 0)),
                  pl.BlockSpec((hid, rd), lambda i: (0, 0)),
                  pl.BlockSpec((hid, out_c), lambda i: (0, 0)),
                  pl.BlockSpec((1, out_c), lambda i: (0, 0))],
        out_specs=[pl.BlockSpec((bn, rd), lambda i: (i, 0)),
                   pl.BlockSpec((bn, out_c), lambda i: (i, 0))],
        out_shape=[jax.ShapeDtypeStruct((n, rd), jnp.float32),
                   jax.ShapeDtypeStruct((n, out_c), jnp.float32)],
    )(p, m0, m1, wcat, root, bias)


def _final_body(p_ref, m0_ref, m1_ref, wc_ref, bc_ref, o_ref):
    h = p_ref[...] + m0_ref[...] + m1_ref[...]
    o_ref[...] = jnp.dot(h, wc_ref[...],
                         preferred_element_type=jnp.float32,
                         precision=_HP) + bc_ref[...]


def _final(p, m0, m1, wc, bc):
    n, out_c = p.shape
    k = wc.shape[1]
    bn = 512
    return pl.pallas_call(
        _final_body,
        grid=(pl.cdiv(n, bn),),
        in_specs=[pl.BlockSpec((bn, out_c), lambda i: (i, 0)),
                  pl.BlockSpec((bn, out_c), lambda i: (i, 0)),
                  pl.BlockSpec((bn, out_c), lambda i: (i, 0)),
                  pl.BlockSpec((out_c, k), lambda i: (0, 0)),
                  pl.BlockSpec((1, k), lambda i: (0, 0))],
        out_specs=pl.BlockSpec((bn, k), lambda i: (i, 0)),
        out_shape=jax.ShapeDtypeStruct((n, k), jnp.float32),
    )(p, m0, m1, wc, bc)


def _count(dst, typ, n_nodes, n_rel):
    e = dst.shape[0]
    rows_per_tile, n_pad = _pad_rows(n_nodes)
    edges_per_tile = e // _NW
    n_chunks = edges_per_tile // _CH
    ngrp = _CH // _L
    ccol = n_rel // _L
    w = 128

    def body(dst_hbm, typ_hbm, out_hbm, dstb, typb, oh, zb, acc):
        core = lax.axis_index("c")
        sub = lax.axis_index("s")
        wid = core * _NUM_SUBCORES + sub
        zeros16 = jnp.zeros((_L,), jnp.float32)
        ones16 = jnp.ones((_L,), jnp.float32)
        iota16 = lax.iota(jnp.int32, _L)
        for i in range(_ZROWS):
            for cc in range(w // _L):
                zb[i, pl.ds(cc * _L, _L)] = zeros16
        for i in range(_CH):
            for cc in range(w // _L):
                oh[i, pl.ds(cc * _L, _L)] = zeros16

        def zloop(i, carry):
            pltpu.sync_copy(
                zb, acc.at[pl.ds(sub * rows_per_tile + i * _ZROWS, _ZROWS)])
            return carry
        lax.fori_loop(0, rows_per_tile // _ZROWS, zloop, 0)
        plsc.subcore_barrier()

        ebase = wid * edges_per_tile

        def chunk(i, carry):
            base = ebase + i * _CH
            pltpu.sync_copy(dst_hbm.at[pl.ds(base, _CH)], dstb)
            pltpu.sync_copy(typ_hbm.at[pl.ds(base, _CH)], typb)

            def mark(g, c2):
                t16 = typb[pl.ds(g * _L, _L)]
                for j in range(_L):
                    e = g * _L + j
                    ts = jnp.full((_L,), t16[j], jnp.int32)
                    for cc in range(ccol):
                        oh[e, pl.ds(cc * _L, _L)] = jnp.where(
                            iota16 + cc * _L == ts, ones16, zeros16)
                return c2
            lax.fori_loop(0, ngrp, mark, 0)
            pltpu.sync_copy(oh, acc.at[dstb], add=True)
            return carry
        lax.fori_loop(0, n_chunks, chunk, 0)
        plsc.subcore_barrier()

        row0 = sub * rows_per_tile
        pltpu.sync_copy(
            acc.at[pl.ds(row0, rows_per_tile)],
            out_hbm.at[pl.ds(core * n_pad + row0, rows_per_tile)])

    return pl.kernel(
        body,
        out_type=jax.ShapeDtypeStruct((2 * n_pad, w), jnp.float32),
        mesh=_mesh(),
        scratch_types=[
            pltpu.VMEM((_CH,), jnp.int32),
            pltpu.VMEM((_CH,), jnp.int32),
            pltpu.VMEM((_CH, w), jnp.float32),
            pltpu.VMEM((_ZROWS, w), jnp.float32),
            pltpu.VMEM_SHARED((n_pad, w), jnp.float32),
        ],
    )(dst, typ)


def _edge_pass(ytab, src, dst, typ, inv, n_nodes, n_rel, d):
    e = src.shape[0]
    rows_per_tile, n_pad = _pad_rows(n_nodes)
    edges_per_tile = e // _NW
    n_chunks = edges_per_tile // _CH
    n_pairs = (n_chunks - 1) // 2
    assert n_chunks == 2 * n_pairs + 1 and n_chunks >= 3
    ngrp = _CH // _L
    dcol = d // _L

    def body(ytab_hbm, src_hbm, dst_hbm, typ_hbm, inv_hbm, out_hbm,
             srcb0, srcb1, dstb0, dstb1, typb0, typb1, keyb0, keyb1,
             adst0, adst1, rows0, rows1, invr0, invr1, zb, acc,
             sl0, sl1, sg, sa0, sa1):
        core = lax.axis_index("c")
        sub = lax.axis_index("s")
        wid = core * _NUM_SUBCORES + sub
        srcbs = (srcb0, srcb1)
        dstbs = (dstb0, dstb1)
        typbs = (typb0, typb1)
        keybs = (keyb0, keyb1)
        adsts = (adst0, adst1)
        rowss = (rows0, rows1)
        invrs = (invr0, invr1)
        sls = (sl0, sl1)
        sas = (sa0, sa1)
        zeros16 = jnp.zeros((_L,), jnp.float32)
        for i in range(_ZROWS):
            for cc in range(dcol):
                zb[i, pl.ds(cc * _L, _L)] = zeros16

        def zloop(i, carry):
            pltpu.sync_copy(
                zb, acc.at[pl.ds(sub * rows_per_tile + i * _ZROWS, _ZROWS)])
            return carry
        lax.fori_loop(0, rows_per_tile // _ZROWS, zloop, 0)
        plsc.subcore_barrier()

        ebase = wid * edges_per_tile

        def issue_load(i, b):
            base = ebase + i * _CH
            pltpu.async_copy(src_hbm.at[pl.ds(base, _CH)], srcbs[b], sls[b])
            pltpu.async_copy(dst_hbm.at[pl.ds(base, _CH)], dstbs[b], sls[b])
            pltpu.async_copy(typ_hbm.at[pl.ds(base, _CH)], typbs[b], sls[b])

        def wait_load(b):
            pltpu.make_async_copy(
                src_hbm.at[pl.ds(ebase, _CH)], srcbs[b], sls[b]).wait()
            pltpu.make_async_copy(
                dst_hbm.at[pl.ds(ebase, _CH)], dstbs[b], sls[b]).wait()
            pltpu.make_async_copy(
                typ_hbm.at[pl.ds(ebase, _CH)], typbs[b], sls[b]).wait()

        def wait_acc(b):
            pltpu.make_async_copy(rowss[b], acc.at[adsts[b]], sas[b]).wait()

        def make_kgrp(b):
            def kgrp(g, c2):
                s16 = srcbs[b][pl.ds(g * _L, _L)]
                t16 = typbs[b][pl.ds(g * _L, _L)]
                keybs[b][pl.ds(g * _L, _L)] = s16 * n_rel + t16
                adsts[b][pl.ds(g * _L, _L)] = dstbs[b][pl.ds(g * _L, _L)]
                return c2
            return kgrp

        def make_sgrp(b):
            rows, invr, typb = rowss[b], invrs[b], typbs[b]

            def sgrp(g, c2):
                t16 = typb[pl.ds(g * _L, _L)]
                for j in range(_L):
                    ee = g * _L + j
                    w = invr[ee, pl.ds(t16[j], _L)][0]
                    ws = jnp.full((_L,), w, jnp.float32)
                    for cc in range(dcol):
                        rows[ee, pl.ds(cc * _L, _L)] = (
                            rows[ee, pl.ds(cc * _L, _L)] * ws)
                return c2
            return sgrp

        def issue_gather(b):
            pltpu.async_copy(ytab_hbm.at[keybs[b]], rowss[b], sg)
            pltpu.async_copy(inv_hbm.at[adsts[b]], invrs[b], sg)

        def wait_gather(b):
            pltpu.make_async_copy(ytab_hbm.at[keybs[b]], rowss[b], sg).wait()
            pltpu.make_async_copy(inv_hbm.at[adsts[b]], invrs[b], sg).wait()

        def process(i, b, steady):
            wait_load(b)
            if steady:
                pl.when(i >= 2)(lambda: wait_acc(b))
            lax.fori_loop(0, ngrp, make_kgrp(b), 0)
            if steady:
                pl.when(i + 1 < n_chunks)(lambda: issue_load(i + 1, 1 - b))
            else:
                issue_load(i + 1, 1 - b)
            issue_gather(b)
            wait_gather(b)
            lax.fori_loop(0, ngrp, make_sgrp(b), 0)
            pltpu.async_copy(rowss[b], acc.at[adsts[b]], sas[b], add=True)

        issue_load(0, 0)
        process(0, 0, False)

        def pair(ii, carry):
            process(2 * ii + 1, 1, True)
            process(2 * ii + 2, 0, True)
            return carry
        lax.fori_loop(0, n_pairs, pair, 0)

        wait_acc(0)
        wait_acc(1)
        plsc.subcore_barrier()

        row0 = sub * rows_per_tile
        pltpu.sync_copy(
            acc.at[pl.ds(row0, rows_per_tile)],
            out_hbm.at[pl.ds(core * n_pad + row0, rows_per_tile)])

    return pl.kernel(
        body,
        out_type=jax.ShapeDtypeStruct((2 * n_pad, d), jnp.float32),
        mesh=_mesh(),
        scratch_types=[
            pltpu.VMEM((_CH,), jnp.int32),
            pltpu.VMEM((_CH,), jnp.int32),
            pltpu.VMEM((_CH,), jnp.int32),
            pltpu.VMEM((_CH,), jnp.int32),
            pltpu.VMEM((_CH,), jnp.int32),
            pltpu.VMEM((_CH,), jnp.int32),
            pltpu.VMEM((_CH,), jnp.int32),
            pltpu.VMEM((_CH,), jnp.int32),
            pltpu.VMEM((_CH,), jnp.int32),
            pltpu.VMEM((_CH,), jnp.int32),
            pltpu.VMEM((_CH, d), jnp.float32),
            pltpu.VMEM((_CH, d), jnp.float32),
            pltpu.VMEM((_CH, 128), jnp.float32),
            pltpu.VMEM((_CH, 128), jnp.float32),
            pltpu.VMEM((_ZROWS, d), jnp.float32),
            pltpu.VMEM_SHARED((n_pad, d), jnp.float32),
            pltpu.SemaphoreType.DMA,
            pltpu.SemaphoreType.DMA,
            pltpu.SemaphoreType.DMA,
            pltpu.SemaphoreType.DMA,
            pltpu.SemaphoreType.DMA,
        ],
    )(ytab, src, dst, typ, inv)


def kernel(x, edge_index, edge_type, comp1, basis1, root1, bias1,
           comp2, basis2, root2, bias2, Wc, bc):
    n, in_c = x.shape
    hid = root1.shape[1]
    out_c = root2.shape[1]
    r, nb = comp1.shape
    ncls = Wc.shape[1]

    src = edge_index[0]
    dst = edge_index[1]

    w1f, w2f = _assemble(comp1, basis1.reshape(nb, in_c * hid),
                         comp2, basis2.reshape(nb, hid * out_c))
    wcat1 = w1f.reshape(r, in_c, hid).transpose(1, 0, 2).reshape(in_c, r * hid)
    wcat2 = w2f.reshape(r, hid, out_c).transpose(1, 0, 2).reshape(hid, r * out_c)

    _, n_pad = _pad_rows(n)
    cnt = _count(dst, edge_type, n, r)
    inv = _inv(cnt.reshape(2, n_pad, 128))

    y1, p1 = _layer_in(x, wcat1, root1, bias1.reshape(1, hid))
    m1 = _edge_pass(y1.reshape(n * r, hid), src, dst, edge_type, inv,
                    n, r, hid)

    y2, p2 = _layer_mid(p1, m1[:n], m1[n_pad:n_pad + n], wcat2, root2,
                        bias2.reshape(1, out_c))
    m2 = _edge_pass(y2.reshape(n * r, out_c), src, dst, edge_type, inv,
                    n, r, out_c)

    wc_pad = jnp.zeros((out_c, 128), jnp.float32).at[:, :ncls].set(Wc)
    bc_pad = jnp.zeros((1, 128), jnp.float32).at[0, :ncls].set(bc)
    of = _final(p2, m2[:n], m2[n_pad:n_pad + n], wc_pad, bc_pad)
    return of[:, :ncls]
